# baseline (speedup 1.0000x reference)
import functools

import jax
import jax.numpy as jnp
from jax import lax
from jax.experimental import pallas as pl
from jax.experimental.pallas import tpu as pltpu
from jax.experimental.pallas import tpu_sc as plsc

N_NODES = 10000
E_EDGES = 160000
D = 256
D_HALF = 128

NC = 2
NS = 16

CHUNK = 128
N_CHUNKS = 80
PER_TILE = N_CHUNKS * CHUNK
E_PAD = NS * PER_TILE
ACC_ROWS = 10240
Z_ROWS = ACC_ROWS // NS
H_CHUNKS = N_CHUNKS // 2
RING = 2

_MM_BM = 5000


def _mm_first_body(x_ref, w_ref, b_ref, o_ref):
    x = x_ref[...]
    w = w_ref[...]
    y = lax.dot_general(x, w, (((1,), (1,)), ((), ())),
                        precision=lax.Precision.DEFAULT,
                        preferred_element_type=jnp.float32)
    o_ref[...] = y + b_ref[0]


def _mm_split_body(x_ref, w_ref, b_ref, o_ref):
    xa = x_ref[0]
    xb = x_ref[1]
    w = w_ref[...]
    ya = lax.dot_general(xa, w[:, :D_HALF], (((1,), (1,)), ((), ())),
                         precision=lax.Precision.DEFAULT,
                         preferred_element_type=jnp.float32)
    yb = lax.dot_general(xb, w[:, D_HALF:], (((1,), (1,)), ((), ())),
                         precision=lax.Precision.DEFAULT,
                         preferred_element_type=jnp.float32)
    o_ref[...] = ya + yb + b_ref[0]


_N_MB = N_NODES // _MM_BM


@jax.jit
def _mm_first(x, w, b2):
    return pl.pallas_call(
        _mm_first_body,
        grid=(NC, _N_MB),
        in_specs=[
            pl.BlockSpec((_MM_BM, D), lambda h, m: (m, 0)),
            pl.BlockSpec((D_HALF, D), lambda h, m: (h, 0)),
            pl.BlockSpec((1, 1, D_HALF), lambda h, m: (h, 0, 0)),
        ],
        out_specs=pl.BlockSpec((_MM_BM, D_HALF), lambda h, m: (h * _N_MB + m, 0)),
        out_shape=jax.ShapeDtypeStruct((NC * N_NODES, D_HALF), jnp.float32),
    )(x, w, b2)


@jax.jit
def _mm_split(x2, w, b2):
    return pl.pallas_call(
        _mm_split_body,
        grid=(NC, _N_MB),
        in_specs=[
            pl.BlockSpec((NC, _MM_BM, D_HALF), lambda h, m: (0, m, 0)),
            pl.BlockSpec((D_HALF, D), lambda h, m: (h, 0)),
            pl.BlockSpec((1, 1, D_HALF), lambda h, m: (h, 0, 0)),
        ],
        out_specs=pl.BlockSpec((_MM_BM, D_HALF), lambda h, m: (h * _N_MB + m, 0)),
        out_shape=jax.ShapeDtypeStruct((NC * N_NODES, D_HALF), jnp.float32),
    )(x2, w, b2)


_sc_mesh = plsc.VectorSubcoreMesh(
    core_axis_name="c", subcore_axis_name="s", num_cores=NC, num_subcores=NS)


_TAIL_ROWS = N_NODES - (NS - 1) * Z_ROWS


def _make_sc_scatter(final):
    if final:
        out_type = jax.ShapeDtypeStruct((N_NODES, D), jnp.float32)
    else:
        out_type = jax.ShapeDtypeStruct((NC * ACC_ROWS, D_HALF), jnp.float32)

    @functools.partial(
        pl.kernel,
        out_type=out_type,
        mesh=_sc_mesh,
        scratch_types=[
            pltpu.VMEM_SHARED((ACC_ROWS, D_HALF), jnp.float32),
            pltpu.VMEM((H_CHUNKS, CHUNK), jnp.int32),
            pltpu.VMEM((H_CHUNKS, CHUNK), jnp.int32),
            pltpu.VMEM((CHUNK, D_HALF), jnp.float32),
            pltpu.VMEM((CHUNK, D_HALF), jnp.float32),
            pltpu.SemaphoreType.DMA,
            pltpu.SemaphoreType.DMA,
        ],
    )
    def _sc_scatter(tbl, src2, dst, zeros, out, acc, sidx, didx,
                    b0, b1, s0, s1):
        bufs = (b0, b1)
        sems = (s0, s1)
        c = lax.axis_index("c")
        s = lax.axis_index("s")

        pltpu.sync_copy(zeros, acc.at[pl.ds(s * Z_ROWS, Z_ROWS)])
        plsc.subcore_barrier()

        for ph in range(2):
            pltpu.sync_copy(
                src2.at[pl.ds((c * NS + s) * N_CHUNKS + ph * H_CHUNKS,
                              H_CHUNKS)],
                sidx)
            pltpu.sync_copy(
                dst.at[pl.ds(s * N_CHUNKS + ph * H_CHUNKS, H_CHUNKS)], didx)

            for p in range(RING - 1):
                pltpu.async_copy(tbl.at[sidx.at[p]], bufs[p], sems[p])

            def block_body(i, carry):
                for b in range(RING):
                    j = i * RING + b
                    nxt = jnp.minimum(j + RING - 1, H_CHUNKS - 1)
                    nb = (b + RING - 1) % RING
                    pltpu.async_copy(tbl.at[sidx.at[nxt]], bufs[nb],
                                     sems[nb])
                    pltpu.make_async_copy(tbl.at[sidx.at[j]], bufs[b],
                                          sems[b]).wait()
                    pltpu.sync_copy(bufs[b], acc.at[didx.at[j]], add=True)
                return carry

            lax.fori_loop(0, H_CHUNKS // RING, block_body, 0)
            for p in range(RING - 1):
                pltpu.make_async_copy(tbl.at[sidx.at[p]], bufs[p],
                                      sems[p]).wait()

        plsc.subcore_barrier()

        row0 = pl.multiple_of(s * Z_ROWS, Z_ROWS)
        if final:
            col0 = pl.multiple_of(c * D_HALF, D_HALF)

            @pl.when(s < NS - 1)
            def _full():
                pltpu.sync_copy(
                    acc.at[pl.ds(row0, Z_ROWS)],
                    out.at[pl.ds(row0, Z_ROWS), pl.ds(col0, D_HALF)])

            @pl.when(s == NS - 1)
            def _tail():
                pltpu.sync_copy(
                    acc.at[pl.ds(row0, _TAIL_ROWS)],
                    out.at[pl.ds(row0, _TAIL_ROWS), pl.ds(col0, D_HALF)])
        else:
            out_row0 = pl.multiple_of(c * ACC_ROWS + s * Z_ROWS, Z_ROWS)
            pltpu.sync_copy(acc.at[pl.ds(row0, Z_ROWS)],
                            out.at[pl.ds(out_row0, Z_ROWS)])

    return _sc_scatter


_sc_scatter = _make_sc_scatter(False)
_sc_scatter_final = _make_sc_scatter(True)


def kernel(X_u, X_v, edge_index, W0, b0, W1, b1, W2, b2):
    ei = edge_index.astype(jnp.int32)
    eu = ei[0]
    ev = ei[1]

    npad = E_PAD - E_EDGES
    pad_ar = jnp.arange(npad, dtype=jnp.int32)
    pad_src = pad_ar % N_NODES
    pad_dst = N_NODES + pad_ar % (ACC_ROWS - N_NODES)

    eu_s = jnp.concatenate([eu, pad_src])
    ev_s = jnp.concatenate([ev, pad_src])
    eu_d = jnp.concatenate([eu, pad_dst]).reshape(NS * N_CHUNKS, CHUNK)
    ev_d = jnp.concatenate([ev, pad_dst]).reshape(NS * N_CHUNKS, CHUNK)

    def two_halves(idx):
        return jnp.concatenate([idx, idx + N_NODES]).reshape(
            NC * NS * N_CHUNKS, CHUNK)

    src_ev = two_halves(ev_s)
    src_eu = two_halves(eu_s)

    zeros = jnp.zeros((Z_ROWS, D_HALF), jnp.float32)
    b0_2 = b0.reshape(NC, 1, D_HALF)
    b1_2 = b1.reshape(NC, 1, D_HALF)
    b2_2 = b2.reshape(NC, 1, D_HALF)

    t0 = _mm_first(X_v, W0, b0_2)
    u1 = _sc_scatter(t0, src_ev, eu_d, zeros)
    t1 = _mm_split(u1.reshape(NC, ACC_ROWS, D_HALF), W1, b1_2)
    v1 = _sc_scatter(t1, src_eu, ev_d, zeros)
    t2 = _mm_split(v1.reshape(NC, ACC_ROWS, D_HALF), W2, b2_2)
    return _sc_scatter_final(t2, src_ev, eu_d, zeros)


# speedup vs baseline: 1.0245x; 1.0245x over previous
import functools

import jax
import jax.numpy as jnp
from jax import lax
from jax.experimental import pallas as pl
from jax.experimental.pallas import tpu as pltpu
from jax.experimental.pallas import tpu_sc as plsc

N_NODES = 10000
E_EDGES = 160000
D = 256
D_HALF = 128

NC = 2
NS = 16

CHUNK = 128
N_CHUNKS = 80
PER_TILE = N_CHUNKS * CHUNK
E_PAD = NS * PER_TILE
ACC_ROWS = 10240
Z_ROWS = ACC_ROWS // NS
H_CHUNKS = N_CHUNKS // 2
RING = 2

_MM_BM = 10000


def _mm_first_body(x_ref, w_ref, b_ref, o_ref):
    x = x_ref[...]
    w = w_ref[...]
    y = lax.dot_general(x, w, (((1,), (1,)), ((), ())),
                        precision=lax.Precision.DEFAULT,
                        preferred_element_type=jnp.float32)
    o_ref[...] = y + b_ref[0]


def _mm_split_body(x_ref, w_ref, b_ref, o_ref):
    xa = x_ref[0]
    xb = x_ref[1]
    w = w_ref[...]
    ya = lax.dot_general(xa, w[:, :D_HALF], (((1,), (1,)), ((), ())),
                         precision=lax.Precision.DEFAULT,
                         preferred_element_type=jnp.float32)
    yb = lax.dot_general(xb, w[:, D_HALF:], (((1,), (1,)), ((), ())),
                         precision=lax.Precision.DEFAULT,
                         preferred_element_type=jnp.float32)
    o_ref[...] = ya + yb + b_ref[0]


_N_MB = N_NODES // _MM_BM


@jax.jit
def _mm_first(x, w, b2):
    return pl.pallas_call(
        _mm_first_body,
        grid=(NC, _N_MB),
        in_specs=[
            pl.BlockSpec((_MM_BM, D), lambda h, m: (m, 0)),
            pl.BlockSpec((D_HALF, D), lambda h, m: (h, 0)),
            pl.BlockSpec((1, 1, D_HALF), lambda h, m: (h, 0, 0)),
        ],
        out_specs=pl.BlockSpec((_MM_BM, D_HALF), lambda h, m: (h * _N_MB + m, 0)),
        out_shape=jax.ShapeDtypeStruct((NC * N_NODES, D_HALF), jnp.float32),
    )(x, w, b2)


@jax.jit
def _mm_split(x2, w, b2):
    return pl.pallas_call(
        _mm_split_body,
        grid=(NC, _N_MB),
        in_specs=[
            pl.BlockSpec((NC, _MM_BM, D_HALF), lambda h, m: (0, m, 0)),
            pl.BlockSpec((D_HALF, D), lambda h, m: (h, 0)),
            pl.BlockSpec((1, 1, D_HALF), lambda h, m: (h, 0, 0)),
        ],
        out_specs=pl.BlockSpec((_MM_BM, D_HALF), lambda h, m: (h * _N_MB + m, 0)),
        out_shape=jax.ShapeDtypeStruct((NC * N_NODES, D_HALF), jnp.float32),
    )(x2, w, b2)


_sc_mesh = plsc.VectorSubcoreMesh(
    core_axis_name="c", subcore_axis_name="s", num_cores=NC, num_subcores=NS)


_TAIL_ROWS = N_NODES - (NS - 1) * Z_ROWS


def _make_sc_scatter(final):
    if final:
        out_type = jax.ShapeDtypeStruct((N_NODES, D), jnp.float32)
    else:
        out_type = jax.ShapeDtypeStruct((NC * ACC_ROWS, D_HALF), jnp.float32)

    @functools.partial(
        pl.kernel,
        out_type=out_type,
        mesh=_sc_mesh,
        scratch_types=[
            pltpu.VMEM_SHARED((ACC_ROWS, D_HALF), jnp.float32),
            pltpu.VMEM((H_CHUNKS, CHUNK), jnp.int32),
            pltpu.VMEM((H_CHUNKS, CHUNK), jnp.int32),
            pltpu.VMEM((CHUNK, D_HALF), jnp.float32),
            pltpu.VMEM((CHUNK, D_HALF), jnp.float32),
            pltpu.SemaphoreType.DMA,
            pltpu.SemaphoreType.DMA,
        ],
    )
    def _sc_scatter(tbl, src2, dst, zeros, out, acc, sidx, didx,
                    b0, b1, s0, s1):
        bufs = (b0, b1)
        sems = (s0, s1)
        c = lax.axis_index("c")
        s = lax.axis_index("s")

        pltpu.sync_copy(zeros, acc.at[pl.ds(s * Z_ROWS, Z_ROWS)])
        plsc.subcore_barrier()

        for ph in range(2):
            pltpu.sync_copy(
                src2.at[pl.ds((c * NS + s) * N_CHUNKS + ph * H_CHUNKS,
                              H_CHUNKS)],
                sidx)
            pltpu.sync_copy(
                dst.at[pl.ds(s * N_CHUNKS + ph * H_CHUNKS, H_CHUNKS)], didx)

            for p in range(RING - 1):
                pltpu.async_copy(tbl.at[sidx.at[p]], bufs[p], sems[p])

            def block_body(i, carry):
                for b in range(RING):
                    j = i * RING + b
                    nxt = jnp.minimum(j + RING - 1, H_CHUNKS - 1)
                    nb = (b + RING - 1) % RING
                    pltpu.async_copy(tbl.at[sidx.at[nxt]], bufs[nb],
                                     sems[nb])
                    pltpu.make_async_copy(tbl.at[sidx.at[j]], bufs[b],
                                          sems[b]).wait()
                    pltpu.sync_copy(bufs[b], acc.at[didx.at[j]], add=True)
                return carry

            lax.fori_loop(0, H_CHUNKS // RING, block_body, 0)
            for p in range(RING - 1):
                pltpu.make_async_copy(tbl.at[sidx.at[p]], bufs[p],
                                      sems[p]).wait()

        plsc.subcore_barrier()

        row0 = pl.multiple_of(s * Z_ROWS, Z_ROWS)
        if final:
            col0 = pl.multiple_of(c * D_HALF, D_HALF)

            @pl.when(s < NS - 1)
            def _full():
                pltpu.sync_copy(
                    acc.at[pl.ds(row0, Z_ROWS)],
                    out.at[pl.ds(row0, Z_ROWS), pl.ds(col0, D_HALF)])

            @pl.when(s == NS - 1)
            def _tail():
                pltpu.sync_copy(
                    acc.at[pl.ds(row0, _TAIL_ROWS)],
                    out.at[pl.ds(row0, _TAIL_ROWS), pl.ds(col0, D_HALF)])
        else:
            out_row0 = pl.multiple_of(c * ACC_ROWS + s * Z_ROWS, Z_ROWS)
            pltpu.sync_copy(acc.at[pl.ds(row0, Z_ROWS)],
                            out.at[pl.ds(out_row0, Z_ROWS)])

    return _sc_scatter


_sc_scatter = _make_sc_scatter(False)
_sc_scatter_final = _make_sc_scatter(True)


def kernel(X_u, X_v, edge_index, W0, b0, W1, b1, W2, b2):
    ei = edge_index.astype(jnp.int32)
    eu = ei[0]
    ev = ei[1]

    npad = E_PAD - E_EDGES
    pad_ar = jnp.arange(npad, dtype=jnp.int32)
    pad_src = pad_ar % N_NODES
    pad_dst = N_NODES + pad_ar % (ACC_ROWS - N_NODES)

    eu_s = jnp.concatenate([eu, pad_src])
    ev_s = jnp.concatenate([ev, pad_src])
    eu_d = jnp.concatenate([eu, pad_dst]).reshape(NS * N_CHUNKS, CHUNK)
    ev_d = jnp.concatenate([ev, pad_dst]).reshape(NS * N_CHUNKS, CHUNK)

    def two_halves(idx):
        return jnp.concatenate([idx, idx + N_NODES]).reshape(
            NC * NS * N_CHUNKS, CHUNK)

    src_ev = two_halves(ev_s)
    src_eu = two_halves(eu_s)

    zeros = jnp.zeros((Z_ROWS, D_HALF), jnp.float32)
    b0_2 = b0.reshape(NC, 1, D_HALF)
    b1_2 = b1.reshape(NC, 1, D_HALF)
    b2_2 = b2.reshape(NC, 1, D_HALF)

    t0 = _mm_first(X_v, W0, b0_2)
    u1 = _sc_scatter(t0, src_ev, eu_d, zeros)
    t1 = _mm_split(u1.reshape(NC, ACC_ROWS, D_HALF), W1, b1_2)
    v1 = _sc_scatter(t1, src_eu, ev_d, zeros)
    t2 = _mm_split(v1.reshape(NC, ACC_ROWS, D_HALF), W2, b2_2)
    return _sc_scatter_final(t2, src_ev, eu_d, zeros)


# speedup vs baseline: 1.0418x; 1.0168x over previous
import functools

import jax
import jax.numpy as jnp
from jax import lax
from jax.experimental import pallas as pl
from jax.experimental.pallas import tpu as pltpu
from jax.experimental.pallas import tpu_sc as plsc

N_NODES = 10000
E_EDGES = 160000
D = 256
D_HALF = 128

NC = 2
NS = 16

CHUNK = 128
N_CHUNKS = 80
PER_TILE = N_CHUNKS * CHUNK
E_PAD = NS * PER_TILE
ACC_ROWS = 10240
Z_ROWS = ACC_ROWS // NS
H_CHUNKS = N_CHUNKS // 2
RING = 2

_MM_BM = 10000


def _mm_first_body(x_ref, w_ref, b_ref, o_ref):
    x = x_ref[...]
    w = w_ref[...]
    y = lax.dot_general(x, w, (((1,), (1,)), ((), ())),
                        precision=lax.Precision.DEFAULT,
                        preferred_element_type=jnp.float32)
    o_ref[...] = y + b_ref[0]


def _mm_split_body(x_ref, w_ref, b_ref, o_ref):
    xa = x_ref[0]
    xb = x_ref[1]
    w = w_ref[...]
    ya = lax.dot_general(xa, w[:, :D_HALF], (((1,), (1,)), ((), ())),
                         precision=lax.Precision.DEFAULT,
                         preferred_element_type=jnp.float32)
    yb = lax.dot_general(xb, w[:, D_HALF:], (((1,), (1,)), ((), ())),
                         precision=lax.Precision.DEFAULT,
                         preferred_element_type=jnp.float32)
    o_ref[...] = ya + yb + b_ref[0]


_N_MB = N_NODES // _MM_BM


@jax.jit
def _mm_first(x, w, b2):
    return pl.pallas_call(
        _mm_first_body,
        grid=(NC, _N_MB),
        in_specs=[
            pl.BlockSpec((_MM_BM, D), lambda h, m: (m, 0)),
            pl.BlockSpec((D_HALF, D), lambda h, m: (h, 0)),
            pl.BlockSpec((1, 1, D_HALF), lambda h, m: (h, 0, 0)),
        ],
        out_specs=pl.BlockSpec((_MM_BM, D_HALF), lambda h, m: (h * _N_MB + m, 0)),
        out_shape=jax.ShapeDtypeStruct((NC * N_NODES, D_HALF), jnp.float32),
    )(x, w, b2)


@jax.jit
def _mm_split(x2, w, b2):
    return pl.pallas_call(
        _mm_split_body,
        grid=(NC, _N_MB),
        in_specs=[
            pl.BlockSpec((NC, _MM_BM, D_HALF), lambda h, m: (0, m, 0)),
            pl.BlockSpec((D_HALF, D), lambda h, m: (h, 0)),
            pl.BlockSpec((1, 1, D_HALF), lambda h, m: (h, 0, 0)),
        ],
        out_specs=pl.BlockSpec((_MM_BM, D_HALF), lambda h, m: (h * _N_MB + m, 0)),
        out_shape=jax.ShapeDtypeStruct((NC * N_NODES, D_HALF), jnp.float32),
    )(x2, w, b2)


_sc_mesh = plsc.VectorSubcoreMesh(
    core_axis_name="c", subcore_axis_name="s", num_cores=NC, num_subcores=NS)


_TAIL_ROWS = N_NODES - (NS - 1) * Z_ROWS


def _make_sc_scatter(final):
    if final:
        out_type = jax.ShapeDtypeStruct((N_NODES, D), jnp.float32)
    else:
        out_type = jax.ShapeDtypeStruct((NC * ACC_ROWS, D_HALF), jnp.float32)

    @functools.partial(
        pl.kernel,
        out_type=out_type,
        mesh=_sc_mesh,
        scratch_types=[
            pltpu.VMEM_SHARED((ACC_ROWS, D_HALF), jnp.float32),
            pltpu.VMEM((H_CHUNKS, CHUNK), jnp.int32),
            pltpu.VMEM((H_CHUNKS, CHUNK), jnp.int32),
            pltpu.VMEM((CHUNK, D_HALF), jnp.float32),
            pltpu.VMEM((CHUNK, D_HALF), jnp.float32),
            pltpu.SemaphoreType.DMA,
            pltpu.SemaphoreType.DMA,
            pltpu.SemaphoreType.DMA,
        ],
    )
    def _sc_scatter(tbl, src2, dst, zeros, out, acc, sidx, didx,
                    b0, b1, s0, s1, zsem):
        bufs = (b0, b1)
        sems = (s0, s1)
        c = lax.axis_index("c")
        s = lax.axis_index("s")

        pltpu.async_copy(zeros, acc.at[pl.ds(s * Z_ROWS, Z_ROWS)], zsem)

        for ph in range(2):
            pltpu.sync_copy(
                src2.at[pl.ds((c * NS + s) * N_CHUNKS + ph * H_CHUNKS,
                              H_CHUNKS)],
                sidx)
            pltpu.sync_copy(
                dst.at[pl.ds(s * N_CHUNKS + ph * H_CHUNKS, H_CHUNKS)], didx)

            for p in range(RING - 1):
                pltpu.async_copy(tbl.at[sidx.at[p]], bufs[p], sems[p])

            if ph == 0:
                pltpu.make_async_copy(
                    zeros, acc.at[pl.ds(s * Z_ROWS, Z_ROWS)], zsem).wait()
                plsc.subcore_barrier()

            def block_body(i, carry):
                for b in range(RING):
                    j = i * RING + b
                    nxt = jnp.minimum(j + RING - 1, H_CHUNKS - 1)
                    nb = (b + RING - 1) % RING
                    pltpu.async_copy(tbl.at[sidx.at[nxt]], bufs[nb],
                                     sems[nb])
                    pltpu.make_async_copy(tbl.at[sidx.at[j]], bufs[b],
                                          sems[b]).wait()
                    pltpu.sync_copy(bufs[b], acc.at[didx.at[j]], add=True)
                return carry

            lax.fori_loop(0, H_CHUNKS // RING, block_body, 0)
            for p in range(RING - 1):
                pltpu.make_async_copy(tbl.at[sidx.at[p]], bufs[p],
                                      sems[p]).wait()

        plsc.subcore_barrier()

        row0 = pl.multiple_of(s * Z_ROWS, Z_ROWS)
        if final:
            col0 = pl.multiple_of(c * D_HALF, D_HALF)

            @pl.when(s < NS - 1)
            def _full():
                pltpu.sync_copy(
                    acc.at[pl.ds(row0, Z_ROWS)],
                    out.at[pl.ds(row0, Z_ROWS), pl.ds(col0, D_HALF)])

            @pl.when(s == NS - 1)
            def _tail():
                pltpu.sync_copy(
                    acc.at[pl.ds(row0, _TAIL_ROWS)],
                    out.at[pl.ds(row0, _TAIL_ROWS), pl.ds(col0, D_HALF)])
        else:
            out_row0 = pl.multiple_of(c * ACC_ROWS + s * Z_ROWS, Z_ROWS)
            pltpu.sync_copy(acc.at[pl.ds(row0, Z_ROWS)],
                            out.at[pl.ds(out_row0, Z_ROWS)])

    return _sc_scatter


_sc_scatter = _make_sc_scatter(False)
_sc_scatter_final = _make_sc_scatter(True)


def kernel(X_u, X_v, edge_index, W0, b0, W1, b1, W2, b2):
    ei = edge_index.astype(jnp.int32)
    eu = ei[0]
    ev = ei[1]

    npad = E_PAD - E_EDGES
    pad_ar = jnp.arange(npad, dtype=jnp.int32)
    pad_src = pad_ar % N_NODES
    pad_dst = N_NODES + pad_ar % (ACC_ROWS - N_NODES)

    eu_s = jnp.concatenate([eu, pad_src])
    ev_s = jnp.concatenate([ev, pad_src])
    eu_d = jnp.concatenate([eu, pad_dst]).reshape(NS * N_CHUNKS, CHUNK)
    ev_d = jnp.concatenate([ev, pad_dst]).reshape(NS * N_CHUNKS, CHUNK)

    def two_halves(idx):
        return jnp.concatenate([idx, idx + N_NODES]).reshape(
            NC * NS * N_CHUNKS, CHUNK)

    src_ev = two_halves(ev_s)
    src_eu = two_halves(eu_s)

    zeros = jnp.zeros((Z_ROWS, D_HALF), jnp.float32)
    b0_2 = b0.reshape(NC, 1, D_HALF)
    b1_2 = b1.reshape(NC, 1, D_HALF)
    b2_2 = b2.reshape(NC, 1, D_HALF)

    t0 = _mm_first(X_v, W0, b0_2)
    u1 = _sc_scatter(t0, src_ev, eu_d, zeros)
    t1 = _mm_split(u1.reshape(NC, ACC_ROWS, D_HALF), W1, b1_2)
    v1 = _sc_scatter(t1, src_eu, ev_d, zeros)
    t2 = _mm_split(v1.reshape(NC, ACC_ROWS, D_HALF), W2, b2_2)
    return _sc_scatter_final(t2, src_ev, eu_d, zeros)
